# baseline (speedup 1.0000x reference)
import functools

import numpy as np
import jax
import jax.numpy as jnp
from jax import lax
from jax.experimental import pallas as pl
from jax.experimental.pallas import tpu as pltpu

_B, _D, _S = 128, 512, 512
_RATE_CONST = 1.0
_EPS = 1e-9
_TINY = np.float32(np.finfo(np.float32).tiny)
_JC = 16
_NB = 4


def _np_threefry_block(k0, k1, x0, x1):
    k0 = np.uint32(k0); k1 = np.uint32(k1)
    ks = [k0, k1, k0 ^ k1 ^ np.uint32(0x1BD11BDA)]
    rots = [(13, 15, 26, 6), (17, 29, 16, 24)]
    x0 = np.uint32(x0) + ks[0]
    x1 = np.uint32(x1) + ks[1]
    for i in range(5):
        for r in rots[i % 2]:
            x0 = np.uint32(x0 + x1)
            x1 = np.uint32((np.uint32(x1 << np.uint32(r))) | (x1 >> np.uint32(32 - r)))
            x1 = x1 ^ x0
        x0 = np.uint32(x0 + ks[(i + 1) % 3])
        x1 = np.uint32(x1 + ks[(i + 2) % 3] + np.uint32(i + 1))
    return x0, x1


_SUBKEYS = [_np_threefry_block(0, 42, np.uint32(0), np.uint32(i)) for i in range(3)]


def _i32(v):
    return jnp.int32(np.int32(np.uint32(v)))


def _tf_bits(kpair, x1):
    k0, k1 = np.uint32(kpair[0]), np.uint32(kpair[1])
    ks = [k0, k1, k0 ^ k1 ^ np.uint32(0x1BD11BDA)]
    rots = [(13, 15, 26, 6), (17, 29, 16, 24)]
    a = jnp.full(x1.shape, _i32(ks[0]), dtype=jnp.int32)
    b = x1 + _i32(ks[1])
    for i in range(5):
        for r in rots[i % 2]:
            a = a + b
            b = lax.shift_left(b, jnp.int32(r)) | lax.shift_right_logical(
                b, jnp.int32(32 - r))
            b = b ^ a
        a = a + _i32(ks[(i + 1) % 3])
        b = b + _i32(np.uint32(ks[(i + 2) % 3] + np.uint32(i + 1)))
    return a ^ b


def _g_of(m):
    fb = m | jnp.int32(0x3F800000)
    fl = lax.bitcast_convert_type(fb, jnp.float32) - jnp.float32(1.0)
    u = jnp.maximum(_TINY, fl + _TINY)
    return -jnp.log(-jnp.log(u))


def _sample_body(xs_ref, bm_ref, bj_ref, mxs_ref):
    g = pl.program_id(0)
    jj_c = lax.broadcasted_iota(jnp.int32, (_JC, _D), 0)
    dd_c = lax.broadcasted_iota(jnp.int32, (_JC, _D), 1)
    big = jnp.int32(1 << 30)

    bases = [(g * _NB + p) * jnp.int32(_D * _S) for p in range(_NB)]
    xss = [xs_ref[p, 0, :] for p in range(_NB)]

    def chunk(c, carry):
        j0 = c * _JC
        jj = jj_c + j0
        off = dd_c * jnp.int32(_S) + jj
        out = []
        for p in range(_NB):
            best_m, best_j, mxs = carry[3 * p:3 * p + 3]
            m = lax.shift_right_logical(_tf_bits(_SUBKEYS[0], bases[p] + off), 9)
            is_xs = jj == xss[p][None, :]
            mm = jnp.where(is_xs, jnp.int32(-1), m)
            cmax = jnp.max(mm, axis=0)
            cj = jnp.min(jnp.where(mm == cmax[None, :], jj, big), axis=0)
            upd = cmax > best_m
            best_m = jnp.where(upd, cmax, best_m)
            best_j = jnp.where(upd, cj, best_j)
            mxs = jnp.maximum(mxs, jnp.max(jnp.where(is_xs, m, jnp.int32(-1)),
                                           axis=0))
            out += [best_m, best_j, mxs]
        return tuple(out)

    init = (jnp.full((_D,), -1, jnp.int32), jnp.full((_D,), 0, jnp.int32),
            jnp.full((_D,), -1, jnp.int32)) * _NB
    carry = lax.fori_loop(0, _S // _JC, chunk, init)

    for p in range(_NB):
        bm_ref[p, 0, :] = carry[3 * p]
        bj_ref[p, 0, :] = carry[3 * p + 1]
        mxs_ref[p, 0, :] = carry[3 * p + 2]


def _finalize_body(xs_ref, lo_ref, hi_ref, bm_ref, bj_ref, mxs_ref, out_ref):
    xs = xs_ref[:, 0, :]
    best_m = bm_ref[:, 0, :]
    best_j = bj_ref[:, 0, :]
    mxs = mxs_ref[:, 0, :]
    big = jnp.int32(1 << 30)

    vstar = _g_of(best_m) + lo_ref[:, 0, :]
    vxs = _g_of(mxs) + hi_ref[:, 0, :]
    xt = jnp.where(vxs > vstar, xs, best_j)
    xt = jnp.where(vxs == vstar, jnp.minimum(xs, best_j), xt)

    rows = lax.broadcasted_iota(jnp.int32, (_B, _D), 0)
    dvec = lax.broadcasted_iota(jnp.int32, (_B, _D), 1)
    f2 = rows * jnp.int32(_D) + dvec
    m2 = lax.shift_right_logical(_tf_bits(_SUBKEYS[1], f2), 9)
    mx2 = jnp.max(m2, axis=1, keepdims=True)
    sd = jnp.min(jnp.where(m2 == mx2, dvec, big), axis=1, keepdims=True)
    xtstar = jnp.max(jnp.where(dvec == sd, xt, jnp.int32(-1)), axis=1,
                     keepdims=True)
    m3 = lax.shift_right_logical(_tf_bits(_SUBKEYS[2], f2), 9)
    m3m = jnp.where(dvec == xtstar, jnp.int32(-1), m3)
    mx3 = jnp.max(m3m, axis=1, keepdims=True)
    nv = jnp.min(jnp.where(m3m == mx3, dvec, big), axis=1, keepdims=True)

    out_ref[...] = jnp.where(dvec == sd, nv, xt)


def _onehot_body(xt_ref, out_ref):
    xt = xt_ref[...]
    sidx = lax.broadcasted_iota(jnp.int32, (8, _B, _S), 2)
    out_ref[...] = (xt[:, :, None] == sidx).astype(jnp.float32)


_NW = 32
_RPW = _D * _B // _NW


def _sc_scatter_body(xt_hbm, out_hbm, xtv, idxv, ones_v):
    from jax.experimental.pallas import tpu_sc as plsc
    wid = lax.axis_index("s") * 2 + lax.axis_index("c")
    rbase = wid * _RPW
    pltpu.sync_copy(xt_hbm.at[pl.ds(rbase * 1, _RPW)], xtv)
    iot = lax.iota(jnp.int32, 16)
    ones16 = jnp.ones((16,), jnp.float32)
    for k in range(8):
        ones_v[pl.ds(k * 16, 16)] = ones16
    for j in range(16):
        for i in range(8):
            r0 = j * 128 + i * 16
            xt16 = xtv[pl.ds(r0, 16)]
            idx16 = (rbase + r0 + iot) * jnp.int32(_S) + xt16
            idxv[j, pl.ds(i * 16, 16)] = idx16
    for j in range(16):
        pltpu.sync_copy(ones_v, out_hbm.at[idxv.at[j]])


def _sc_scatter(xt_flat, buf_ref):
    from jax.experimental.pallas import tpu_sc as plsc
    mesh = plsc.VectorSubcoreMesh(core_axis_name="c", subcore_axis_name="s")
    k = pl.kernel(
        _sc_scatter_body,
        out_type=(),
        mesh=mesh,
        scratch_types=[
            pltpu.VMEM((_RPW,), jnp.int32),
            pltpu.VMEM((16, 128), jnp.int32),
            pltpu.VMEM((128,), jnp.float32),
        ],
    )
    k(xt_flat, buf_ref)


@jax.jit
def kernel(x_start, t):
    xs3 = x_start.astype(jnp.int32).reshape(_B, 1, _D)
    w = jnp.exp(-_S * _RATE_CONST * t)
    lo = jnp.log((1.0 - w) / _S + _EPS).astype(jnp.float32)
    hi = jnp.log(w + (1.0 - w) / _S + _EPS).astype(jnp.float32)
    lo3 = jnp.broadcast_to(lo[:, None, None], (_B, 1, _D))
    hi3 = jnp.broadcast_to(hi[:, None, None], (_B, 1, _D))

    i3 = jax.ShapeDtypeStruct((_B, 1, _D), jnp.int32)
    bspec = pl.BlockSpec((_NB, 1, _D), lambda b: (b, 0, 0))
    bm, bj, mxs = pl.pallas_call(
        _sample_body,
        grid=(_B // _NB,),
        in_specs=[bspec],
        out_specs=(bspec, bspec, bspec),
        out_shape=(i3, i3, i3),
    )(xs3)

    full = pl.BlockSpec((_B, 1, _D), lambda: (0, 0, 0))
    xtilde = pl.pallas_call(
        _finalize_body,
        grid=(),
        in_specs=[full] * 6,
        out_specs=pl.BlockSpec((_B, _D), lambda: (0, 0)),
        out_shape=jax.ShapeDtypeStruct((_B, _D), jnp.int32),
    )(xs3, lo3, hi3, bm, bj, mxs)

    xt_T = jnp.transpose(xtilde)

    buf = jax.new_ref(jnp.zeros((_D * _B * _S,), jnp.float32))
    _sc_scatter(xt_T.reshape(-1), buf)
    return buf[...].reshape(_D, _B, _S)


# speedup vs baseline: 1.3499x; 1.3499x over previous
import functools

import numpy as np
import jax
import jax.numpy as jnp
from jax import lax
from jax.experimental import pallas as pl
from jax.experimental.pallas import tpu as pltpu

_B, _D, _S = 128, 512, 512
_RATE_CONST = 1.0
_EPS = 1e-9
_TINY = np.float32(np.finfo(np.float32).tiny)
_JC = 16
_NB = 4


def _np_threefry_block(k0, k1, x0, x1):
    k0 = np.uint32(k0); k1 = np.uint32(k1)
    ks = [k0, k1, k0 ^ k1 ^ np.uint32(0x1BD11BDA)]
    rots = [(13, 15, 26, 6), (17, 29, 16, 24)]
    x0 = np.uint32(x0) + ks[0]
    x1 = np.uint32(x1) + ks[1]
    for i in range(5):
        for r in rots[i % 2]:
            x0 = np.uint32(x0 + x1)
            x1 = np.uint32((np.uint32(x1 << np.uint32(r))) | (x1 >> np.uint32(32 - r)))
            x1 = x1 ^ x0
        x0 = np.uint32(x0 + ks[(i + 1) % 3])
        x1 = np.uint32(x1 + ks[(i + 2) % 3] + np.uint32(i + 1))
    return x0, x1


_SUBKEYS = [_np_threefry_block(0, 42, np.uint32(0), np.uint32(i)) for i in range(3)]


def _i32(v):
    return jnp.int32(np.int32(np.uint32(v)))


def _tf_bits(kpair, x1, prebiased=False):
    k0, k1 = np.uint32(kpair[0]), np.uint32(kpair[1])
    ks = [k0, k1, k0 ^ k1 ^ np.uint32(0x1BD11BDA)]
    rots = [(13, 15, 26, 6), (17, 29, 16, 24)]
    a = jnp.full(x1.shape, _i32(ks[0]), dtype=jnp.int32)
    b = x1 if prebiased else x1 + _i32(ks[1])
    for i in range(5):
        for r in rots[i % 2]:
            a = a + b
            b = lax.shift_left(b, jnp.int32(r)) | lax.shift_right_logical(
                b, jnp.int32(32 - r))
            b = b ^ a
        a = a + _i32(ks[(i + 1) % 3])
        b = b + _i32(np.uint32(ks[(i + 2) % 3] + np.uint32(i + 1)))
    return a ^ b


def _g_of(m):
    fb = m | jnp.int32(0x3F800000)
    fl = lax.bitcast_convert_type(fb, jnp.float32) - jnp.float32(1.0)
    u = jnp.maximum(_TINY, fl + _TINY)
    return -jnp.log(-jnp.log(u))


_SIGN = np.int32(np.uint32(0x80000000))
_HIMASK = np.int32(np.uint32(0xFFFFFE00))
_IMIN = np.int32(-2**31)


def _sample_body(xs_ref, bk_ref, xk_ref):
    g = pl.program_id(0)
    jj_c = lax.broadcasted_iota(jnp.int32, (_JC, _D), 0)
    dd_c = lax.broadcasted_iota(jnp.int32, (_JC, _D), 1)
    off_c = dd_c * jnp.int32(_S) + jj_c
    rj_c = jnp.int32(_S - 1) - jj_c

    ks1 = np.uint32(_SUBKEYS[0][1])
    bases = [(g * _NB + p) * jnp.int32(_D * _S) + _i32(ks1)
             for p in range(_NB)]
    xss = [xs_ref[p, 0, :] for p in range(_NB)]
    sign = jnp.int32(_SIGN)
    himask = jnp.int32(_HIMASK)
    imin = jnp.int32(_IMIN)

    def chunk(c, carry):
        j0 = c * _JC
        jj = jj_c + j0
        off = off_c + j0
        rj = rj_c - j0
        out = []
        for p in range(_NB):
            bk, xk = carry[2 * p:2 * p + 2]
            bits = _tf_bits(_SUBKEYS[0], bases[p] + off, prebiased=True)
            key = ((bits ^ sign) & himask) | rj
            is_xs = jj == xss[p][None, :]
            bk = jnp.maximum(bk, jnp.max(jnp.where(is_xs, imin, key), axis=0))
            xk = jnp.maximum(xk, jnp.max(jnp.where(is_xs, key, imin), axis=0))
            out += [bk, xk]
        return tuple(out)

    init = (jnp.full((_D,), _IMIN, jnp.int32),
            jnp.full((_D,), _IMIN, jnp.int32)) * _NB
    carry = lax.fori_loop(0, _S // _JC, chunk, init)

    for p in range(_NB):
        bk_ref[p, 0, :] = carry[2 * p]
        xk_ref[p, 0, :] = carry[2 * p + 1]


def _finalize_body(xs_ref, lo_ref, hi_ref, bk_ref, xk_ref, out_ref):
    xs = xs_ref[:, 0, :]
    sign = jnp.int32(_SIGN)
    bku = bk_ref[:, 0, :] ^ sign
    best_m = lax.shift_right_logical(bku, 9)
    best_j = jnp.int32(_S - 1) - (bku & jnp.int32(_S - 1))
    mxs = lax.shift_right_logical(xk_ref[:, 0, :] ^ sign, 9)
    big = jnp.int32(1 << 30)

    vstar = _g_of(best_m) + lo_ref[:, 0, :]
    vxs = _g_of(mxs) + hi_ref[:, 0, :]
    xt = jnp.where(vxs > vstar, xs, best_j)
    xt = jnp.where(vxs == vstar, jnp.minimum(xs, best_j), xt)

    rows = lax.broadcasted_iota(jnp.int32, (_B, _D), 0)
    dvec = lax.broadcasted_iota(jnp.int32, (_B, _D), 1)
    f2 = rows * jnp.int32(_D) + dvec
    m2 = lax.shift_right_logical(_tf_bits(_SUBKEYS[1], f2), 9)
    mx2 = jnp.max(m2, axis=1, keepdims=True)
    sd = jnp.min(jnp.where(m2 == mx2, dvec, big), axis=1, keepdims=True)
    xtstar = jnp.max(jnp.where(dvec == sd, xt, jnp.int32(-1)), axis=1,
                     keepdims=True)
    m3 = lax.shift_right_logical(_tf_bits(_SUBKEYS[2], f2), 9)
    m3m = jnp.where(dvec == xtstar, jnp.int32(-1), m3)
    mx3 = jnp.max(m3m, axis=1, keepdims=True)
    nv = jnp.min(jnp.where(m3m == mx3, dvec, big), axis=1, keepdims=True)

    out_ref[...] = jnp.where(dvec == sd, nv, xt)


def _onehot_body(xt_ref, out_ref):
    xt = xt_ref[...]
    sidx = lax.broadcasted_iota(jnp.int32, (8, _B, _S), 2)
    out_ref[...] = (xt[:, :, None] == sidx).astype(jnp.float32)


@jax.jit
def kernel(x_start, t):
    xs3 = x_start.astype(jnp.int32).reshape(_B, 1, _D)
    w = jnp.exp(-_S * _RATE_CONST * t)
    lo = jnp.log((1.0 - w) / _S + _EPS).astype(jnp.float32)
    hi = jnp.log(w + (1.0 - w) / _S + _EPS).astype(jnp.float32)
    lo3 = jnp.broadcast_to(lo[:, None, None], (_B, 1, _D))
    hi3 = jnp.broadcast_to(hi[:, None, None], (_B, 1, _D))

    i3 = jax.ShapeDtypeStruct((_B, 1, _D), jnp.int32)
    bspec = pl.BlockSpec((_NB, 1, _D), lambda b: (b, 0, 0))
    bk, xk = pl.pallas_call(
        _sample_body,
        grid=(_B // _NB,),
        in_specs=[bspec],
        out_specs=(bspec, bspec),
        out_shape=(i3, i3),
    )(xs3)

    full = pl.BlockSpec((_B, 1, _D), lambda: (0, 0, 0))
    xtilde = pl.pallas_call(
        _finalize_body,
        grid=(),
        in_specs=[full] * 5,
        out_specs=pl.BlockSpec((_B, _D), lambda: (0, 0)),
        out_shape=jax.ShapeDtypeStruct((_B, _D), jnp.int32),
    )(xs3, lo3, hi3, bk, xk)

    xt_T = jnp.transpose(xtilde)

    out = pl.pallas_call(
        _onehot_body,
        grid=(_D // 8,),
        in_specs=[pl.BlockSpec((8, _B), lambda i: (i, 0))],
        out_specs=pl.BlockSpec((8, _B, _S), lambda i: (i, 0, 0)),
        out_shape=jax.ShapeDtypeStruct((_D, _B, _S), jnp.float32),
    )(xt_T)
    return out


# speedup vs baseline: 1.3649x; 1.0112x over previous
import functools

import numpy as np
import jax
import jax.numpy as jnp
from jax import lax
from jax.experimental import pallas as pl
from jax.experimental.pallas import tpu as pltpu

_B, _D, _S = 128, 512, 512
_RATE_CONST = 1.0
_EPS = 1e-9
_TINY = np.float32(np.finfo(np.float32).tiny)
_JC = 32
_NB = 2


def _np_threefry_block(k0, k1, x0, x1):
    k0 = np.uint32(k0); k1 = np.uint32(k1)
    ks = [k0, k1, k0 ^ k1 ^ np.uint32(0x1BD11BDA)]
    rots = [(13, 15, 26, 6), (17, 29, 16, 24)]
    x0 = np.uint32(x0) + ks[0]
    x1 = np.uint32(x1) + ks[1]
    for i in range(5):
        for r in rots[i % 2]:
            x0 = np.uint32(x0 + x1)
            x1 = np.uint32((np.uint32(x1 << np.uint32(r))) | (x1 >> np.uint32(32 - r)))
            x1 = x1 ^ x0
        x0 = np.uint32(x0 + ks[(i + 1) % 3])
        x1 = np.uint32(x1 + ks[(i + 2) % 3] + np.uint32(i + 1))
    return x0, x1


_SUBKEYS = [_np_threefry_block(0, 42, np.uint32(0), np.uint32(i)) for i in range(3)]


def _i32(v):
    return jnp.int32(np.int32(np.uint32(v)))


def _tf_bits(kpair, x1, prebiased=False):
    k0, k1 = np.uint32(kpair[0]), np.uint32(kpair[1])
    ks = [k0, k1, k0 ^ k1 ^ np.uint32(0x1BD11BDA)]
    rots = [(13, 15, 26, 6), (17, 29, 16, 24)]
    a = jnp.full(x1.shape, _i32(ks[0]), dtype=jnp.int32)
    b = x1 if prebiased else x1 + _i32(ks[1])
    for i in range(5):
        for r in rots[i % 2]:
            a = a + b
            b = lax.shift_left(b, jnp.int32(r)) | lax.shift_right_logical(
                b, jnp.int32(32 - r))
            b = b ^ a
        a = a + _i32(ks[(i + 1) % 3])
        b = b + _i32(np.uint32(ks[(i + 2) % 3] + np.uint32(i + 1)))
    return a ^ b


def _g_of(m):
    fb = m | jnp.int32(0x3F800000)
    fl = lax.bitcast_convert_type(fb, jnp.float32) - jnp.float32(1.0)
    u = jnp.maximum(_TINY, fl + _TINY)
    return -jnp.log(-jnp.log(u))


_SIGN = np.int32(np.uint32(0x80000000))
_HIMASK = np.int32(np.uint32(0xFFFFFE00))
_IMIN = np.int32(-2**31)


def _sample_body(xs_ref, bk_ref, xk_ref):
    g = pl.program_id(0)
    jj_c = lax.broadcasted_iota(jnp.int32, (_JC, _D), 0)
    dd_c = lax.broadcasted_iota(jnp.int32, (_JC, _D), 1)
    off_c = dd_c * jnp.int32(_S) + jj_c
    rj_c = jnp.int32(_S - 1) - jj_c

    ks1 = np.uint32(_SUBKEYS[0][1])
    bases = [(g * _NB + p) * jnp.int32(_D * _S) + _i32(ks1)
             for p in range(_NB)]
    xss = [xs_ref[p, 0, :] for p in range(_NB)]
    sign = jnp.int32(_SIGN)
    himask = jnp.int32(_HIMASK)
    imin = jnp.int32(_IMIN)

    def chunk(c, carry):
        j0 = c * _JC
        jj = jj_c + j0
        off = off_c + j0
        rj = rj_c - j0
        out = []
        for p in range(_NB):
            bk, xk = carry[2 * p:2 * p + 2]
            bits = _tf_bits(_SUBKEYS[0], bases[p] + off, prebiased=True)
            key = ((bits ^ sign) & himask) | rj
            is_xs = jj == xss[p][None, :]
            bk = jnp.maximum(bk, jnp.max(jnp.where(is_xs, imin, key), axis=0))
            xk = jnp.maximum(xk, jnp.max(jnp.where(is_xs, key, imin), axis=0))
            out += [bk, xk]
        return tuple(out)

    init = (jnp.full((_D,), _IMIN, jnp.int32),
            jnp.full((_D,), _IMIN, jnp.int32)) * _NB
    carry = lax.fori_loop(0, _S // _JC, chunk, init)

    for p in range(_NB):
        bk_ref[p, 0, :] = carry[2 * p]
        xk_ref[p, 0, :] = carry[2 * p + 1]


def _finalize_body(xs_ref, lo_ref, hi_ref, bk_ref, xk_ref, out_ref):
    xs = xs_ref[:, 0, :]
    sign = jnp.int32(_SIGN)
    bku = bk_ref[:, 0, :] ^ sign
    best_m = lax.shift_right_logical(bku, 9)
    best_j = jnp.int32(_S - 1) - (bku & jnp.int32(_S - 1))
    mxs = lax.shift_right_logical(xk_ref[:, 0, :] ^ sign, 9)
    big = jnp.int32(1 << 30)

    vstar = _g_of(best_m) + lo_ref[:, 0, :]
    vxs = _g_of(mxs) + hi_ref[:, 0, :]
    xt = jnp.where(vxs > vstar, xs, best_j)
    xt = jnp.where(vxs == vstar, jnp.minimum(xs, best_j), xt)

    rows = lax.broadcasted_iota(jnp.int32, (_B, _D), 0)
    dvec = lax.broadcasted_iota(jnp.int32, (_B, _D), 1)
    f2 = rows * jnp.int32(_D) + dvec
    m2 = lax.shift_right_logical(_tf_bits(_SUBKEYS[1], f2), 9)
    mx2 = jnp.max(m2, axis=1, keepdims=True)
    sd = jnp.min(jnp.where(m2 == mx2, dvec, big), axis=1, keepdims=True)
    xtstar = jnp.max(jnp.where(dvec == sd, xt, jnp.int32(-1)), axis=1,
                     keepdims=True)
    m3 = lax.shift_right_logical(_tf_bits(_SUBKEYS[2], f2), 9)
    m3m = jnp.where(dvec == xtstar, jnp.int32(-1), m3)
    mx3 = jnp.max(m3m, axis=1, keepdims=True)
    nv = jnp.min(jnp.where(m3m == mx3, dvec, big), axis=1, keepdims=True)

    out_ref[...] = jnp.where(dvec == sd, nv, xt)


def _onehot_body(xt_ref, out_ref):
    xt = xt_ref[...]
    sidx = lax.broadcasted_iota(jnp.int32, (8, _B, _S), 2)
    out_ref[...] = (xt[:, :, None] == sidx).astype(jnp.float32)


@jax.jit
def kernel(x_start, t):
    xs3 = x_start.astype(jnp.int32).reshape(_B, 1, _D)
    w = jnp.exp(-_S * _RATE_CONST * t)
    lo = jnp.log((1.0 - w) / _S + _EPS).astype(jnp.float32)
    hi = jnp.log(w + (1.0 - w) / _S + _EPS).astype(jnp.float32)
    lo3 = jnp.broadcast_to(lo[:, None, None], (_B, 1, _D))
    hi3 = jnp.broadcast_to(hi[:, None, None], (_B, 1, _D))

    i3 = jax.ShapeDtypeStruct((_B, 1, _D), jnp.int32)
    bspec = pl.BlockSpec((_NB, 1, _D), lambda b: (b, 0, 0))
    bk, xk = pl.pallas_call(
        _sample_body,
        grid=(_B // _NB,),
        in_specs=[bspec],
        out_specs=(bspec, bspec),
        out_shape=(i3, i3),
    )(xs3)

    full = pl.BlockSpec((_B, 1, _D), lambda: (0, 0, 0))
    xtilde = pl.pallas_call(
        _finalize_body,
        grid=(),
        in_specs=[full] * 5,
        out_specs=pl.BlockSpec((_B, _D), lambda: (0, 0)),
        out_shape=jax.ShapeDtypeStruct((_B, _D), jnp.int32),
    )(xs3, lo3, hi3, bk, xk)

    xt_T = jnp.transpose(xtilde)

    out = pl.pallas_call(
        _onehot_body,
        grid=(_D // 8,),
        in_specs=[pl.BlockSpec((8, _B), lambda i: (i, 0))],
        out_specs=pl.BlockSpec((8, _B, _S), lambda i: (i, 0, 0)),
        out_shape=jax.ShapeDtypeStruct((_D, _B, _S), jnp.float32),
    )(xt_T)
    return out
